# Initial kernel scaffold; baseline (speedup 1.0000x reference)
import jax
import jax.numpy as jnp
from jax.experimental import pallas as pl


def kernel(points):
    raise NotImplementedError("write your pallas kernel here")


# speedup vs baseline: 1.7548x; 1.7548x over previous
import jax
import jax.numpy as jnp
from jax.experimental import pallas as pl

NUM_GROUPS = 512
GROUP_SIZE = 32


def _fps_body(xp_ref, start_ref, sel_ref, cx_ref, cy_ref, cz_ref):
    xs = xp_ref[0]
    ys = xp_ref[1]
    zs = xp_ref[2]
    B, N = xs.shape
    G = NUM_GROUPS
    start = start_ref[...]
    iota = jax.lax.broadcasted_iota(jnp.int32, (B, N), 1)
    giota = jax.lax.broadcasted_iota(jnp.int32, (B, G), 1)

    oh0 = iota == start
    lx = jnp.sum(jnp.where(oh0, xs, 0.0), axis=1, keepdims=True)
    ly = jnp.sum(jnp.where(oh0, ys, 0.0), axis=1, keepdims=True)
    lz = jnp.sum(jnp.where(oh0, zs, 0.0), axis=1, keepdims=True)

    sel_acc = jnp.where(giota == 0, start, 0)
    cx_acc = jnp.where(giota == 0, lx, 0.0)
    cy_acc = jnp.where(giota == 0, ly, 0.0)
    cz_acc = jnp.where(giota == 0, lz, 0.0)
    dists = jnp.full((B, N), jnp.inf, dtype=jnp.float32)

    def body(k, carry):
        lx, ly, lz, dists, sel_acc, cx_acc, cy_acc, cz_acc = carry
        dx = xs - lx
        dy = ys - ly
        dz = zs - lz
        d = (dx * dx + dy * dy) + dz * dz
        dists = jnp.minimum(dists, d)
        m = jnp.max(dists, axis=1, keepdims=True)
        idx = jnp.min(jnp.where(dists == m, iota, N), axis=1, keepdims=True)
        ohk = iota == idx
        lx = jnp.sum(jnp.where(ohk, xs, 0.0), axis=1, keepdims=True)
        ly = jnp.sum(jnp.where(ohk, ys, 0.0), axis=1, keepdims=True)
        lz = jnp.sum(jnp.where(ohk, zs, 0.0), axis=1, keepdims=True)
        ohg = giota == k
        sel_acc = jnp.where(ohg, idx, sel_acc)
        cx_acc = jnp.where(ohg, lx, cx_acc)
        cy_acc = jnp.where(ohg, ly, cy_acc)
        cz_acc = jnp.where(ohg, lz, cz_acc)
        return (lx, ly, lz, dists, sel_acc, cx_acc, cy_acc, cz_acc)

    carry = (lx, ly, lz, dists, sel_acc, cx_acc, cy_acc, cz_acc)
    carry = jax.lax.fori_loop(1, G, body, carry)
    _, _, _, _, sel_acc, cx_acc, cy_acc, cz_acc = carry
    sel_ref[...] = sel_acc
    cx_ref[...] = cx_acc
    cy_ref[...] = cy_acc
    cz_ref[...] = cz_acc


def _fps_pallas(xp, starts):
    B = xp.shape[1]
    G = NUM_GROUPS
    out_shapes = (
        jax.ShapeDtypeStruct((B, G), jnp.int32),
        jax.ShapeDtypeStruct((B, G), jnp.float32),
        jax.ShapeDtypeStruct((B, G), jnp.float32),
        jax.ShapeDtypeStruct((B, G), jnp.float32),
    )
    return pl.pallas_call(
        _fps_body,
        out_shape=out_shapes,
    )(xp, starts)


def kernel(points):
    B, N, C = points.shape
    xyz = points[:, :, :3]
    xp = jnp.moveaxis(xyz, 2, 0)
    starts = jax.random.randint(jax.random.key(42), (B,), 0, N).astype(jnp.int32)
    sel, cx, cy, cz = _fps_pallas(xp, starts.reshape(B, 1))
    centers = jnp.stack([cx, cy, cz], axis=-1)

    cn = jnp.sum(centers ** 2, axis=-1)
    pn = jnp.sum(xyz ** 2, axis=-1)
    cross = jnp.einsum('bgd,bnd->bgn', centers, xyz)
    d2 = cn[:, :, None] - 2.0 * cross + pn[:, None, :]
    _, idx = jax.lax.top_k(-d2, GROUP_SIZE)
    grouped = jax.vmap(lambda p, i: p[i])(points, idx)
    grouped = jnp.concatenate(
        [grouped[..., :3] - centers[:, :, None, :], grouped[..., 3:]], axis=-1)
    return grouped, centers


# speedup vs baseline: 11.5868x; 6.6031x over previous
import jax
import jax.numpy as jnp
from jax import lax
from jax.experimental import pallas as pl
from jax.experimental.pallas import tpu as pltpu
from jax.experimental.pallas import tpu_sc as plsc

NUM_GROUPS = 512
GROUP_SIZE = 32
_NW = 32
_RPW = 128
_CAP = 528
_BIG = 1 << 30


def _fps_body(xp_ref, start_ref, sel_ref, cx_ref, cy_ref, cz_ref):
    xs = xp_ref[0]
    ys = xp_ref[1]
    zs = xp_ref[2]
    B, N = xs.shape
    G = NUM_GROUPS
    start = start_ref[...]
    iota = jax.lax.broadcasted_iota(jnp.int32, (B, N), 1)
    giota = jax.lax.broadcasted_iota(jnp.int32, (B, G), 1)

    oh0 = iota == start
    lx = jnp.sum(jnp.where(oh0, xs, 0.0), axis=1, keepdims=True)
    ly = jnp.sum(jnp.where(oh0, ys, 0.0), axis=1, keepdims=True)
    lz = jnp.sum(jnp.where(oh0, zs, 0.0), axis=1, keepdims=True)

    sel_acc = jnp.where(giota == 0, start, 0)
    cx_acc = jnp.where(giota == 0, lx, 0.0)
    cy_acc = jnp.where(giota == 0, ly, 0.0)
    cz_acc = jnp.where(giota == 0, lz, 0.0)
    dists = jnp.full((B, N), jnp.inf, dtype=jnp.float32)

    def body(k, carry):
        lx, ly, lz, dists, sel_acc, cx_acc, cy_acc, cz_acc = carry
        dx = xs - lx
        dy = ys - ly
        dz = zs - lz
        d = (dx * dx + dz * dz) + dy * dy
        dists = jnp.minimum(dists, d)
        m = jnp.max(dists, axis=1, keepdims=True)
        idx = jnp.min(jnp.where(dists == m, iota, N), axis=1, keepdims=True)
        ohk = iota == idx
        lx = jnp.sum(jnp.where(ohk, xs, 0.0), axis=1, keepdims=True)
        ly = jnp.sum(jnp.where(ohk, ys, 0.0), axis=1, keepdims=True)
        lz = jnp.sum(jnp.where(ohk, zs, 0.0), axis=1, keepdims=True)
        ohg = giota == k
        sel_acc = jnp.where(ohg, idx, sel_acc)
        cx_acc = jnp.where(ohg, lx, cx_acc)
        cy_acc = jnp.where(ohg, ly, cy_acc)
        cz_acc = jnp.where(ohg, lz, cz_acc)
        return (lx, ly, lz, dists, sel_acc, cx_acc, cy_acc, cz_acc)

    carry = (lx, ly, lz, dists, sel_acc, cx_acc, cy_acc, cz_acc)
    carry = jax.lax.fori_loop(1, G, body, carry)
    _, _, _, _, sel_acc, cx_acc, cy_acc, cz_acc = carry
    sel_ref[...] = sel_acc
    cx_ref[...] = cx_acc
    cy_ref[...] = cy_acc
    cz_ref[...] = cz_acc


def _fps_pallas(xp, starts):
    B = xp.shape[1]
    G = NUM_GROUPS
    out_shapes = (
        jax.ShapeDtypeStruct((B, G), jnp.int32),
        jax.ShapeDtypeStruct((B, G), jnp.float32),
        jax.ShapeDtypeStruct((B, G), jnp.float32),
        jax.ShapeDtypeStruct((B, G), jnp.float32),
    )
    return pl.pallas_call(
        _fps_body,
        out_shape=out_shapes,
    )(xp, starts)


def _d2_body(c_ref, xt_ref, d2_ref, cm_ref):
    c = c_ref[0]
    xt = xt_ref[0]
    GB = c.shape[0]
    cross = jax.lax.dot_general(
        c, xt, (((1,), (0,)), ((), ())), preferred_element_type=jnp.float32)
    c0 = c[:, 0:1]
    c1 = c[:, 1:2]
    c2 = c[:, 2:3]
    cn = (c0 * c0 + c1 * c1) + c2 * c2
    xs = xt[0:1, :]
    ys = xt[1:2, :]
    zs = xt[2:3, :]
    pn = (xs * xs + ys * ys) + zs * zs
    d2 = (cn - 2.0 * cross) + pn
    d2_ref[0] = d2
    cm128 = jnp.min(d2.reshape(GB, 64, 128), axis=1)
    cm_ref[0] = jnp.minimum(cm128[:, :64], cm128[:, 64:])


def _d2_pallas(centers, xT):
    B, G, _ = centers.shape
    N = xT.shape[2]
    GB = 256
    return pl.pallas_call(
        _d2_body,
        grid=(B, G // GB),
        in_specs=[
            pl.BlockSpec((1, GB, 3), lambda b, j: (b, j, 0)),
            pl.BlockSpec((1, 3, N), lambda b, j: (b, 0, 0)),
        ],
        out_specs=[
            pl.BlockSpec((1, GB, N), lambda b, j: (b, j, 0)),
            pl.BlockSpec((1, GB, 64), lambda b, j: (b, j, 0)),
        ],
        out_shape=(
            jax.ShapeDtypeStruct((B, G, N), jnp.float32),
            jax.ShapeDtypeStruct((B, G, 64), jnp.float32),
        ),
    )(centers, xT)


def _srt(v):
    k, _ = plsc.sort_key_val(v, v)
    return k


def _merge16(a, b):
    rb = lax.rev(b, (0,))
    return _srt(jnp.minimum(a, rb)), _srt(jnp.maximum(a, rb))


def _sc_body(d2_hbm, cm_hbm, pts_hbm, ctr_hbm, out_hbm,
             ptsv, cmv, ctrv, rowbuf, sidx, svals, outv, rsem):
    caxis = lax.axis_index("c")
    saxis = lax.axis_index("s")
    w = saxis * 2 + caxis
    b = w // 4
    base = w * _RPW
    iota = lax.iota(jnp.int32, 16)
    inf = jnp.float32(jnp.inf)
    N = 8192

    pltpu.sync_copy(pts_hbm.at[pl.ds(b * N * 6, N * 6)], ptsv)
    pltpu.sync_copy(cm_hbm.at[pl.ds(base * 64, _RPW * 64)], cmv)
    pltpu.sync_copy(ctr_hbm.at[pl.ds(base * 3, _RPW * 3)], ctrv)

    def initb(j, _):
        sidx[pl.ds(j * 16, 16)] = jnp.zeros((16,), jnp.int32)
        return 0

    lax.fori_loop(0, _CAP // 16, initb, 0)

    pltpu.async_copy(d2_hbm.at[pl.ds(base * N, N)], rowbuf.at[pl.ds(0, N)], rsem)
    pltpu.async_copy(d2_hbm.at[pl.ds((base + 1) * N, N)],
                     rowbuf.at[pl.ds(N, N)], rsem)

    def process(g, slot):
        pltpu.make_async_copy(
            d2_hbm.at[pl.ds((base + g) * N, N)],
            rowbuf.at[pl.ds(slot * N, N)], rsem).wait()

        s0 = _srt(cmv[pl.ds(g * 64, 16)])
        s1 = _srt(cmv[pl.ds(g * 64 + 16, 16)])
        s2 = _srt(cmv[pl.ds(g * 64 + 32, 16)])
        s3 = _srt(cmv[pl.ds(g * 64 + 48, 16)])
        m0, m1 = _merge16(s0, s1)
        m2, m3 = _merge16(s2, s3)
        lo0 = jnp.minimum(m0, lax.rev(m3, (0,)))
        lo1 = jnp.minimum(m1, lax.rev(m2, (0,)))
        t = jnp.max(jnp.maximum(lo0, lo1))
        tb = jnp.broadcast_to(t, (16,))

        def p2(i, off):
            v = rowbuf[pl.ds(slot * N + i * 16, 16)]
            msk = v <= tb
            ones = jnp.where(msk, 1, 0).astype(jnp.int32)
            pos = off + plsc.cumsum(ones) - 1
            msk2 = jnp.logical_and(msk, pos < _CAP)
            plsc.store_scatter(sidx, [pos], iota + i * 16, mask=msk2)
            return off + plsc.all_reduce_population_count(msk)

        off = lax.fori_loop(0, 512, p2, jnp.zeros((16,), jnp.int32), unroll=4)
        cnt = jnp.max(off)
        cntb = jnp.broadcast_to(cnt, (16,))
        nv = (cnt + 15) // 16

        sbase = jnp.full((16,), slot * N, jnp.int32)

        def gsv(j, _):
            siv = sidx[pl.ds(j * 16, 16)]
            valid = (iota + j * 16) < cntb
            vv = plsc.load_gather(rowbuf, [sbase + siv], mask=valid)
            svals[pl.ds(j * 16, 16)] = jnp.where(valid, vv, inf)
            return 0

        lax.fori_loop(0, nv, gsv, 0)

        def emk(k, st):
            wv0, wv1, prev = st

            def pass1(j, acc):
                vv = svals[pl.ds(j * 16, 16)]
                si = sidx[pl.ds(j * 16, 16)]
                vv = jnp.where(si == prev, inf, vv)
                svals[pl.ds(j * 16, 16)] = vv
                return jnp.minimum(acc, vv)

            acc = lax.fori_loop(0, nv, pass1, jnp.full((16,), inf))
            mb = jnp.broadcast_to(jnp.min(acc), (16,))

            def pass2(j, acc2):
                vv = svals[pl.ds(j * 16, 16)]
                si = sidx[pl.ds(j * 16, 16)]
                return jnp.minimum(acc2, jnp.where(vv == mb, si, _BIG))

            acc2 = lax.fori_loop(0, nv, pass2,
                                 jnp.full((16,), _BIG, jnp.int32))
            ib = jnp.broadcast_to(jnp.minimum(jnp.min(acc2), 8191), (16,))
            wv0 = jnp.where(iota == k, ib, wv0)
            wv1 = jnp.where(iota == (k - 16), ib, wv1)
            return (wv0, wv1, ib)

        zero16 = jnp.zeros((16,), jnp.int32)
        wv0, wv1, _ = lax.fori_loop(
            0, 32, emk, (zero16, zero16, jnp.full((16,), -1, jnp.int32)))

        orow = g % 16
        coff = g * 3
        csplat = [plsc.load_gather(ctrv, [jnp.full((16,), ch, jnp.int32) + coff])
                  for ch in range(3)]
        for vi, wv in enumerate((wv0, wv1)):
            kvec = iota + vi * 16
            for ch in range(6):
                val = plsc.load_gather(ptsv, [wv * 6 + ch])
                if ch < 3:
                    val = val - csplat[ch]
                plsc.store_scatter(outv, [orow * 192 + kvec * 6 + ch], val)

        @pl.when(orow == 15)
        def _():
            pltpu.sync_copy(
                outv, out_hbm.at[pl.ds((base + g - 15) * 192, 16 * 192)])

        @pl.when(g + 2 < _RPW)
        def _():
            pltpu.async_copy(d2_hbm.at[pl.ds((base + g + 2) * N, N)],
                             rowbuf.at[pl.ds(slot * N, N)], rsem)

    def outer(gp, _):
        process(gp * 2, 0)
        process(gp * 2 + 1, 1)
        return 0

    lax.fori_loop(0, _RPW // 2, outer, 0)


def _select_gather_sc(d2, cm, points, centers):
    B, G, N = d2.shape
    R = B * G
    mesh = plsc.VectorSubcoreMesh(
        core_axis_name="c", subcore_axis_name="s", num_cores=2, num_subcores=16)
    run = pl.kernel(
        _sc_body,
        out_type=jax.ShapeDtypeStruct((R * GROUP_SIZE * 6,), jnp.float32),
        mesh=mesh,
        compiler_params=pltpu.CompilerParams(needs_layout_passes=False),
        scratch_types=[
            pltpu.VMEM((N * 6,), jnp.float32),
            pltpu.VMEM((_RPW * 64,), jnp.float32),
            pltpu.VMEM((_RPW * 3,), jnp.float32),
            pltpu.VMEM((2 * N,), jnp.float32),
            pltpu.VMEM((_CAP,), jnp.int32),
            pltpu.VMEM((_CAP,), jnp.float32),
            pltpu.VMEM((16 * GROUP_SIZE * 6,), jnp.float32),
            pltpu.SemaphoreType.DMA,
        ],
    )
    grouped = run(d2.reshape(R * N), cm.reshape(R * 64), points.reshape(-1),
                  centers.reshape(R * 3))
    return grouped.reshape(B, G, GROUP_SIZE, 6)


def kernel(points):
    B, N, C = points.shape
    xyz = points[:, :, :3]
    xp = jnp.moveaxis(xyz, 2, 0)
    starts = jax.random.randint(jax.random.key(42), (B,), 0, N).astype(jnp.int32)
    sel, cx, cy, cz = _fps_pallas(xp, starts.reshape(B, 1))
    centers = jnp.stack([cx, cy, cz], axis=-1)

    xT = jnp.swapaxes(xyz, 1, 2)
    d2, cm = _d2_pallas(centers, xT)

    grouped = _select_gather_sc(d2, cm, points, centers)
    return grouped, centers


# speedup vs baseline: 26.3142x; 2.2710x over previous
import jax
import jax.numpy as jnp
from jax import lax
from jax.experimental import pallas as pl
from jax.experimental.pallas import tpu as pltpu
from jax.experimental.pallas import tpu_sc as plsc

NUM_GROUPS = 512
GROUP_SIZE = 32
_NW = 32
_RPW = 128
_CAP = 528
_BIG = 1 << 30


def _fps_body(xp_ref, start_ref, sel_ref, cx_ref, cy_ref, cz_ref):
    xs = xp_ref[0]
    ys = xp_ref[1]
    zs = xp_ref[2]
    B, N = xs.shape
    G = NUM_GROUPS
    start = start_ref[...]
    iota = jax.lax.broadcasted_iota(jnp.int32, (B, N), 1)
    giota = jax.lax.broadcasted_iota(jnp.int32, (B, G), 1)

    oh0 = iota == start
    lx = jnp.sum(jnp.where(oh0, xs, 0.0), axis=1, keepdims=True)
    ly = jnp.sum(jnp.where(oh0, ys, 0.0), axis=1, keepdims=True)
    lz = jnp.sum(jnp.where(oh0, zs, 0.0), axis=1, keepdims=True)

    sel_acc = jnp.where(giota == 0, start, 0)
    cx_acc = jnp.where(giota == 0, lx, 0.0)
    cy_acc = jnp.where(giota == 0, ly, 0.0)
    cz_acc = jnp.where(giota == 0, lz, 0.0)
    dists = jnp.full((B, N), jnp.inf, dtype=jnp.float32)

    def body(k, carry):
        lx, ly, lz, dists, sel_acc, cx_acc, cy_acc, cz_acc = carry
        dx = xs - lx
        dy = ys - ly
        dz = zs - lz
        d = (dx * dx + dz * dz) + dy * dy
        dists = jnp.minimum(dists, d)
        m = jnp.max(dists, axis=1, keepdims=True)
        idx = jnp.min(jnp.where(dists == m, iota, N), axis=1, keepdims=True)
        ohk = iota == idx
        lx = jnp.sum(jnp.where(ohk, xs, 0.0), axis=1, keepdims=True)
        ly = jnp.sum(jnp.where(ohk, ys, 0.0), axis=1, keepdims=True)
        lz = jnp.sum(jnp.where(ohk, zs, 0.0), axis=1, keepdims=True)
        ohg = giota == k
        sel_acc = jnp.where(ohg, idx, sel_acc)
        cx_acc = jnp.where(ohg, lx, cx_acc)
        cy_acc = jnp.where(ohg, ly, cy_acc)
        cz_acc = jnp.where(ohg, lz, cz_acc)
        return (lx, ly, lz, dists, sel_acc, cx_acc, cy_acc, cz_acc)

    carry = (lx, ly, lz, dists, sel_acc, cx_acc, cy_acc, cz_acc)
    carry = jax.lax.fori_loop(1, G, body, carry)
    _, _, _, _, sel_acc, cx_acc, cy_acc, cz_acc = carry
    sel_ref[...] = sel_acc
    cx_ref[...] = cx_acc
    cy_ref[...] = cy_acc
    cz_ref[...] = cz_acc


def _fps_pallas(xp, starts):
    B = xp.shape[1]
    G = NUM_GROUPS
    out_shapes = (
        jax.ShapeDtypeStruct((B, G), jnp.int32),
        jax.ShapeDtypeStruct((B, G), jnp.float32),
        jax.ShapeDtypeStruct((B, G), jnp.float32),
        jax.ShapeDtypeStruct((B, G), jnp.float32),
    )
    return pl.pallas_call(
        _fps_body,
        out_shape=out_shapes,
    )(xp, starts)


def _d2_body(c_ref, xt_ref, d2_ref, cm_ref):
    c = c_ref[0]
    xt = xt_ref[0]
    GB = c.shape[0]
    cross = jax.lax.dot_general(
        c, xt, (((1,), (0,)), ((), ())), preferred_element_type=jnp.float32)
    c0 = c[:, 0:1]
    c1 = c[:, 1:2]
    c2 = c[:, 2:3]
    cn = (c0 * c0 + c1 * c1) + c2 * c2
    xs = xt[0:1, :]
    ys = xt[1:2, :]
    zs = xt[2:3, :]
    pn = (xs * xs + ys * ys) + zs * zs
    d2 = (cn - 2.0 * cross) + pn
    d2_ref[0] = d2
    cm128 = jnp.min(d2.reshape(GB, 64, 128), axis=1)
    cm_ref[0] = jnp.minimum(cm128[:, :64], cm128[:, 64:])


def _d2_pallas(centers, xT):
    B, G, _ = centers.shape
    N = xT.shape[2]
    GB = 256
    return pl.pallas_call(
        _d2_body,
        grid=(B, G // GB),
        in_specs=[
            pl.BlockSpec((1, GB, 3), lambda b, j: (b, j, 0)),
            pl.BlockSpec((1, 3, N), lambda b, j: (b, 0, 0)),
        ],
        out_specs=[
            pl.BlockSpec((1, GB, N), lambda b, j: (b, j, 0)),
            pl.BlockSpec((1, GB, 64), lambda b, j: (b, j, 0)),
        ],
        out_shape=(
            jax.ShapeDtypeStruct((B, G, N), jnp.float32),
            jax.ShapeDtypeStruct((B, G, 64), jnp.float32),
        ),
    )(centers, xT)


def _srt(v):
    k, _ = plsc.sort_key_val(v, v)
    return k


def _merge16(a, b):
    rb = lax.rev(b, (0,))
    return _srt(jnp.minimum(a, rb)), _srt(jnp.maximum(a, rb))


def _sc_body(d2_hbm, cm_hbm, pts_hbm, ctr_hbm, out_hbm,
             ptsv, cmv, ctrv, rowbuf, sidx, svals, outv, rsem):
    caxis = lax.axis_index("c")
    saxis = lax.axis_index("s")
    w = saxis * 2 + caxis
    b = w // 4
    base = w * _RPW
    iota = lax.iota(jnp.int32, 16)
    inf = jnp.float32(jnp.inf)
    N = 8192

    pltpu.sync_copy(pts_hbm.at[pl.ds(b * N * 6, N * 6)], ptsv)
    pltpu.sync_copy(cm_hbm.at[pl.ds(base * 64, _RPW * 64)], cmv)
    pltpu.sync_copy(ctr_hbm.at[pl.ds(base * 3, _RPW * 3)], ctrv)

    def initb(j, _):
        sidx[pl.ds(j * 16, 16)] = jnp.zeros((16,), jnp.int32)
        return 0

    lax.fori_loop(0, _CAP // 16, initb, 0)

    pltpu.async_copy(d2_hbm.at[pl.ds(base * N, N)], rowbuf.at[pl.ds(0, N)], rsem)
    pltpu.async_copy(d2_hbm.at[pl.ds((base + 1) * N, N)],
                     rowbuf.at[pl.ds(N, N)], rsem)

    def process(g, slot):
        pltpu.make_async_copy(
            d2_hbm.at[pl.ds((base + g) * N, N)],
            rowbuf.at[pl.ds(slot * N, N)], rsem).wait()

        s0 = _srt(cmv[pl.ds(g * 64, 16)])
        s1 = _srt(cmv[pl.ds(g * 64 + 16, 16)])
        s2 = _srt(cmv[pl.ds(g * 64 + 32, 16)])
        s3 = _srt(cmv[pl.ds(g * 64 + 48, 16)])
        m0, m1 = _merge16(s0, s1)
        m2, m3 = _merge16(s2, s3)
        lo0 = jnp.minimum(m0, lax.rev(m3, (0,)))
        lo1 = jnp.minimum(m1, lax.rev(m2, (0,)))
        t = jnp.max(jnp.maximum(lo0, lo1))
        tb = jnp.broadcast_to(t, (16,))

        @plsc.parallel_loop(0, 512, carry=jnp.zeros((16,), jnp.int32),
                            unroll=8)
        def p2(i, off):
            v = rowbuf[pl.ds(slot * N + i * 16, 16)]
            msk = v <= tb
            ones = jnp.where(msk, 1, 0).astype(jnp.int32)
            pos = off + plsc.cumsum(ones) - 1
            msk2 = jnp.logical_and(msk, pos < _CAP)
            plsc.store_scatter(sidx, [pos], iota + i * 16, mask=msk2)
            return off + plsc.all_reduce_population_count(msk)

        cnt = jnp.minimum(jnp.max(p2), _CAP)
        cntb = jnp.broadcast_to(cnt, (16,))
        nv = (cnt + 15) // 16

        sbase = jnp.full((16,), slot * N, jnp.int32)

        def gsv(j, _):
            siv = sidx[pl.ds(j * 16, 16)]
            valid = (iota + j * 16) < cntb
            vv = plsc.load_gather(rowbuf, [sbase + siv], mask=valid)
            svals[pl.ds(j * 16, 16)] = jnp.where(valid, vv, inf)
            return 0

        lax.fori_loop(0, jnp.maximum(nv, 4), gsv, 0)

        def skv(k, i2):
            return plsc.sort_key_val(k, i2)

        def pmm(ak, ai, bk, bi):
            m = jnp.logical_or(ak < bk,
                               jnp.logical_and(ak == bk, ai < bi))
            return (jnp.where(m, ak, bk), jnp.where(m, ai, bi),
                    jnp.where(m, bk, ak), jnp.where(m, bi, ai))

        def ld(j):
            return svals[pl.ds(j * 16, 16)], sidx[pl.ds(j * 16, 16)]

        def rv(k, i2):
            return lax.rev(k, (0,)), lax.rev(i2, (0,))

        def merge32(ak, ai, bk, bi):
            rbk, rbi = rv(bk, bi)
            lok, loi, hik, hii = pmm(ak, ai, rbk, rbi)
            lok, loi = skv(lok, loi)
            hik, hii = skv(hik, hii)
            return lok, loi, hik, hii

        k0, i0 = ld(0)
        k1, i1 = ld(1)
        k2, i2 = ld(2)
        k3, i3 = ld(3)
        k0, i0 = skv(k0, i0)
        k1, i1 = skv(k1, i1)
        k2, i2 = skv(k2, i2)
        k3, i3 = skv(k3, i3)
        a0k, a0i, a1k, a1i = merge32(k0, i0, k1, i1)
        b0k, b0i, b1k, b1i = merge32(k2, i2, k3, i3)
        r1k, r1i = rv(b1k, b1i)
        r0k, r0i = rv(b0k, b0i)
        l0k, l0i, _, _ = pmm(a0k, a0i, r1k, r1i)
        l1k, l1i, _, _ = pmm(a1k, a1i, r0k, r0i)
        lo0k, lo0i, hi0k, hi0i = pmm(l0k, l0i, l1k, l1i)
        w0k, w0i = skv(lo0k, lo0i)
        w1k, w1i = skv(hi0k, hi0i)

        def fold(j, st):
            w0k, w0i, w1k, w1i = st
            ek, ei = ld(j)
            ek, ei = skv(ek, ei)
            rek, rei = rv(ek, ei)
            mlk, mli, _, _ = pmm(w1k, w1i, rek, rei)
            lok, loi, hik, hii = pmm(w0k, w0i, mlk, mli)
            w0k, w0i = skv(lok, loi)
            w1k, w1i = skv(hik, hii)
            return (w0k, w0i, w1k, w1i)

        w0k, w0i, w1k, w1i = lax.fori_loop(
            4, jnp.maximum(nv, 4), fold, (w0k, w0i, w1k, w1i))
        wv0 = jnp.minimum(w0i, 8191)
        wv1 = jnp.minimum(w1i, 8191)

        orow = g % 16
        coff = g * 3
        csplat = [plsc.load_gather(ctrv, [jnp.full((16,), ch, jnp.int32) + coff])
                  for ch in range(3)]
        for vi, wv in enumerate((wv0, wv1)):
            kvec = iota + vi * 16
            for ch in range(6):
                val = plsc.load_gather(ptsv, [wv * 6 + ch])
                if ch < 3:
                    val = val - csplat[ch]
                plsc.store_scatter(outv, [orow * 192 + kvec * 6 + ch], val)

        @pl.when(orow == 15)
        def _():
            pltpu.sync_copy(
                outv, out_hbm.at[pl.ds((base + g - 15) * 192, 16 * 192)])

        @pl.when(g + 2 < _RPW)
        def _():
            pltpu.async_copy(d2_hbm.at[pl.ds((base + g + 2) * N, N)],
                             rowbuf.at[pl.ds(slot * N, N)], rsem)

    def outer(gp, _):
        process(gp * 2, 0)
        process(gp * 2 + 1, 1)
        return 0

    lax.fori_loop(0, _RPW // 2, outer, 0)


def _select_gather_sc(d2, cm, points, centers):
    B, G, N = d2.shape
    R = B * G
    mesh = plsc.VectorSubcoreMesh(
        core_axis_name="c", subcore_axis_name="s", num_cores=2, num_subcores=16)
    run = pl.kernel(
        _sc_body,
        out_type=jax.ShapeDtypeStruct((R * GROUP_SIZE * 6,), jnp.float32),
        mesh=mesh,
        compiler_params=pltpu.CompilerParams(needs_layout_passes=False),
        scratch_types=[
            pltpu.VMEM((N * 6,), jnp.float32),
            pltpu.VMEM((_RPW * 64,), jnp.float32),
            pltpu.VMEM((_RPW * 3,), jnp.float32),
            pltpu.VMEM((2 * N,), jnp.float32),
            pltpu.VMEM((_CAP,), jnp.int32),
            pltpu.VMEM((_CAP,), jnp.float32),
            pltpu.VMEM((16 * GROUP_SIZE * 6,), jnp.float32),
            pltpu.SemaphoreType.DMA,
        ],
    )
    grouped = run(d2.reshape(R * N), cm.reshape(R * 64), points.reshape(-1),
                  centers.reshape(R * 3))
    return grouped.reshape(B, G, GROUP_SIZE, 6)


def kernel(points):
    B, N, C = points.shape
    xyz = points[:, :, :3]
    xp = jnp.moveaxis(xyz, 2, 0)
    starts = jax.random.randint(jax.random.key(42), (B,), 0, N).astype(jnp.int32)
    sel, cx, cy, cz = _fps_pallas(xp, starts.reshape(B, 1))
    centers = jnp.stack([cx, cy, cz], axis=-1)

    xT = jnp.swapaxes(xyz, 1, 2)
    d2, cm = _d2_pallas(centers, xT)

    grouped = _select_gather_sc(d2, cm, points, centers)
    return grouped, centers
